# baseline (speedup 1.0000x reference)
import functools

import jax
import jax.numpy as jnp
from jax import lax
from jax.experimental import pallas as pl
from jax.experimental.pallas import tpu as pltpu
from jax.experimental.pallas import tpu_sc as plsc

_N = 10000
_E = 320000
_D = 128
_DEPTH = 4
_EPS = 1e-5

_NC = 2
_NS = 16
_NW = _NC * _NS
_EPW = _E // _NW
_CH = 104
_NCHUNK = 97
_EPWP = _NCHUNK * _CH
_TRASH = _N
_NR = _N + 8
_NZFULL = 96

_mesh = plsc.VectorSubcoreMesh(core_axis_name="c", subcore_axis_name="s")

@functools.partial(
    pl.kernel,
    out_type=jax.ShapeDtypeStruct((_NC * _N, _D), jnp.float32),
    mesh=_mesh,
    scratch_types=[
        pltpu.VMEM((_NCHUNK, _CH), jnp.int32),
        pltpu.VMEM((_CH, _D), jnp.float32),
        pltpu.VMEM((_CH, _D), jnp.float32),
        pltpu.VMEM_SHARED((_NR, _D), jnp.float32),
    ],
)
def _deg_kernel(dst_hbm, out_hbm, dst_v, ones_v, zeros_v, acc_s):
    cid = lax.axis_index("c")
    sid = lax.axis_index("s")
    wid = sid * _NC + cid

    def fill(i, carry):
        for k in range(_D // 16):
            ones_v[i, pl.ds(k * 16, 16)] = jnp.ones((16,), jnp.float32)
            zeros_v[i, pl.ds(k * 16, 16)] = jnp.zeros((16,), jnp.float32)
        return carry
    lax.fori_loop(0, _CH, fill, 0)
    for t in range(pl.cdiv(_NZFULL + 1, _NS)):
        c = sid + t * _NS
        @pl.when(c < _NZFULL)
        def _zero():
            pltpu.sync_copy(zeros_v, acc_s.at[pl.ds(c * _CH, _CH)])
        @pl.when(c == _NZFULL)
        def _ztail():
            pltpu.sync_copy(zeros_v.at[pl.ds(0, _NR - _NZFULL * _CH)],
                            acc_s.at[pl.ds(_NZFULL * _CH,
                                           _NR - _NZFULL * _CH)])
    pltpu.sync_copy(dst_hbm.at[wid], dst_v)
    plsc.subcore_barrier()
    def body(j, carry):
        pltpu.sync_copy(ones_v, acc_s.at[dst_v.at[j]], add=True)
        return carry
    lax.fori_loop(0, _NCHUNK, body, 0)
    plsc.subcore_barrier()
    @pl.when(sid < 10)
    def _dump():
        pltpu.sync_copy(acc_s.at[pl.ds(sid * 1000, 1000)],
                        out_hbm.at[pl.ds(cid * _N + sid * 1000, 1000)])


@functools.partial(
    pl.kernel,
    out_type=jax.ShapeDtypeStruct((_NC * _N, _D), jnp.float32),
    mesh=_mesh,
    scratch_types=[
        pltpu.VMEM((_EPWP,), jnp.int32),
        pltpu.VMEM((_NCHUNK, _CH), jnp.int32),
        pltpu.VMEM((_CH, _D), jnp.float32),
        pltpu.VMEM((_CH, _D), jnp.float32),
        pltpu.VMEM_SHARED((_NR, _D), jnp.float32),
        pltpu.SemaphoreType.DMA,
        pltpu.SemaphoreType.DMA,
    ],
)
def _agg_kernel(xs_hbm, src_hbm, dst_hbm, out_hbm,
                src_v, dst_v, buf0, buf1, acc_s, sem0, sem1):
    cid = lax.axis_index("c")
    sid = lax.axis_index("s")
    wid = sid * _NC + cid
    def zrow(i, carry):
        for k in range(_D // 16):
            buf0[i, pl.ds(k * 16, 16)] = jnp.zeros((16,), jnp.float32)
        return carry
    lax.fori_loop(0, _CH, zrow, 0)
    for t in range(pl.cdiv(_NZFULL + 1, _NS)):
        c = sid + t * _NS
        @pl.when(c < _NZFULL)
        def _zero():
            pltpu.sync_copy(buf0, acc_s.at[pl.ds(c * _CH, _CH)])
        @pl.when(c == _NZFULL)
        def _ztail():
            pltpu.sync_copy(buf0.at[pl.ds(0, _NR - _NZFULL * _CH)],
                            acc_s.at[pl.ds(_NZFULL * _CH,
                                           _NR - _NZFULL * _CH)])
    pltpu.sync_copy(src_hbm.at[wid], src_v)
    pltpu.sync_copy(dst_hbm.at[wid], dst_v)
    plsc.subcore_barrier()
    def src_at(j):
        return src_v.at[pl.ds(j * _CH, _CH)]

    pltpu.async_copy(xs_hbm.at[src_at(0)], buf0, sem0)

    def body(jj, carry):
        j0 = 2 * jj
        pltpu.make_async_copy(xs_hbm.at[src_at(j0)], buf0, sem0).wait()
        @pl.when(j0 + 1 < _NCHUNK)
        def _g1():
            pltpu.async_copy(xs_hbm.at[src_at(j0 + 1)], buf1, sem1)
        pltpu.sync_copy(buf0, acc_s.at[dst_v.at[j0]], add=True)
        @pl.when(j0 + 1 < _NCHUNK)
        def _s1():
            pltpu.make_async_copy(xs_hbm.at[src_at(j0 + 1)], buf1, sem1).wait()
            @pl.when(j0 + 2 < _NCHUNK)
            def _g0():
                pltpu.async_copy(xs_hbm.at[src_at(j0 + 2)], buf0, sem0)
            pltpu.sync_copy(buf1, acc_s.at[dst_v.at[j0 + 1]], add=True)
        return carry
    lax.fori_loop(0, pl.cdiv(_NCHUNK, 2), body, 0)
    plsc.subcore_barrier()
    @pl.when(sid < 10)
    def _dump():
        pltpu.sync_copy(acc_s.at[pl.ds(sid * 1000, 1000)],
                        out_hbm.at[pl.ds(cid * _N + sid * 1000, 1000)])


def _tc_first(x_ref, w_ref, deg_ref, xs_ref, dis_ref):
    deg = deg_ref[0, :, 0:1] + deg_ref[1, :, 0:1]
    dis = jnp.where(deg > 0, lax.rsqrt(deg), 0.0)
    dis_ref[...] = dis
    xw = jnp.dot(x_ref[...], w_ref[...], preferred_element_type=jnp.float32)
    xs_ref[...] = dis * xw


def _post_layer(agg_ref, dis, b, gamma, beta, h_prev):
    s = agg_ref[0] + agg_ref[1]
    pre = dis * s + b[None, :]
    mu = jnp.mean(pre, axis=0, keepdims=True)
    var = jnp.mean((pre - mu) ** 2, axis=0, keepdims=True)
    hb = gamma[None, :] * (pre - mu) / jnp.sqrt(var + _EPS) + beta[None, :]
    return jnp.maximum(hb, 0.0) + h_prev


def _tc_mid(agg_ref, dis_ref, b_ref, g_ref, bt_ref, h_ref, w_ref,
            h_out_ref, xs_ref):
    dis = dis_ref[...]
    h_new = _post_layer(agg_ref, dis, b_ref[...], g_ref[...], bt_ref[...],
                        h_ref[...])
    h_out_ref[...] = h_new
    xw = jnp.dot(h_new, w_ref[...], preferred_element_type=jnp.float32)
    xs_ref[...] = dis * xw


def _tc_last(agg_ref, dis_ref, b_ref, g_ref, bt_ref, h_ref,
             rw0_ref, rb0_ref, rw1_ref, rb1_ref, rw2_ref, rb2_ref, y_ref):
    h_new = _post_layer(agg_ref, dis_ref[...], b_ref[...], g_ref[...],
                        bt_ref[...], h_ref[...])
    g = jnp.mean(h_new, axis=0, keepdims=True)
    y = jnp.dot(g, rw0_ref[...], preferred_element_type=jnp.float32)
    y = jnp.maximum(y + rb0_ref[...][None, :], 0.0)
    y = jnp.dot(y, rw1_ref[...], preferred_element_type=jnp.float32)
    y = jnp.maximum(y + rb1_ref[...][None, :], 0.0)
    y = jnp.dot(y, rw2_ref[...], preferred_element_type=jnp.float32)
    y_ref[...] = y + rb2_ref[...][None, :]


_f32 = jnp.float32


def kernel(x, edge_index, W, b, gamma, beta, rW0, rb0, rW1, rb1, rW2, rb2):
    pad = _EPWP - _EPW
    src2 = jnp.pad(edge_index[0].reshape(_NW, _EPW), ((0, 0), (0, pad)))
    dst3 = jnp.pad(edge_index[1].reshape(_NW, _EPW), ((0, 0), (0, pad)),
                   constant_values=_TRASH).reshape(_NW, _NCHUNK, _CH)

    deg2 = _deg_kernel(dst3).reshape(_NC, _N, _D)

    xs, dis = pl.pallas_call(
        _tc_first,
        out_shape=(jax.ShapeDtypeStruct((_N, _D), _f32),
                   jax.ShapeDtypeStruct((_N, 1), _f32)),
    )(x, W[0], deg2)

    h = x
    for i in range(_DEPTH - 1):
        agg2 = _agg_kernel(xs, src2, dst3).reshape(_NC, _N, _D)
        h, xs = pl.pallas_call(
            _tc_mid,
            out_shape=(jax.ShapeDtypeStruct((_N, _D), _f32),
                       jax.ShapeDtypeStruct((_N, _D), _f32)),
        )(agg2, dis, b[i], gamma[i], beta[i], h, W[i + 1])

    agg2 = _agg_kernel(xs, src2, dst3).reshape(_NC, _N, _D)
    y = pl.pallas_call(
        _tc_last,
        out_shape=jax.ShapeDtypeStruct((1, len(rb2)), _f32),
    )(agg2, dis, b[_DEPTH - 1], gamma[_DEPTH - 1], beta[_DEPTH - 1], h,
      rW0, rb0, rW1, rb1, rW2, rb2)
    return y.reshape(-1)


# speedup vs baseline: 1.4292x; 1.4292x over previous
import functools

import jax
import jax.numpy as jnp
from jax import lax
from jax.experimental import pallas as pl
from jax.experimental.pallas import tpu as pltpu
from jax.experimental.pallas import tpu_sc as plsc

_N = 10000
_E = 320000
_D = 128
_DEPTH = 4
_EPS = 1e-5

_NC = 2
_NS = 16
_NW = _NC * _NS
_EPW = _E // _NW
_CH = 80
_NCHUNK = _EPW // _CH
_EPWP = _NCHUNK * _CH
_NR = _N
_NZFULL = _N // _CH

_mesh = plsc.VectorSubcoreMesh(core_axis_name="c", subcore_axis_name="s")

@functools.partial(
    pl.kernel,
    out_type=jax.ShapeDtypeStruct((_NC * _N, _D), jnp.float32),
    mesh=_mesh,
    scratch_types=[
        pltpu.VMEM((_NCHUNK, _CH), jnp.int32),
        pltpu.VMEM((_CH, _D), jnp.float32),
        pltpu.VMEM((_CH, _D), jnp.float32),
        pltpu.VMEM_SHARED((_NR, _D), jnp.float32),
    ],
)
def _deg_kernel(dst_hbm, out_hbm, dst_v, ones_v, zeros_v, acc_s):
    cid = lax.axis_index("c")
    sid = lax.axis_index("s")
    wid = sid * _NC + cid

    def fill(i, carry):
        for k in range(_D // 16):
            ones_v[i, pl.ds(k * 16, 16)] = jnp.ones((16,), jnp.float32)
            zeros_v[i, pl.ds(k * 16, 16)] = jnp.zeros((16,), jnp.float32)
        return carry
    lax.fori_loop(0, _CH, fill, 0)
    for t in range(pl.cdiv(_NZFULL, _NS)):
        c = sid + t * _NS
        @pl.when(c < _NZFULL)
        def _zero():
            pltpu.sync_copy(zeros_v, acc_s.at[pl.ds(c * _CH, _CH)])
    pltpu.sync_copy(dst_hbm.at[wid], dst_v)
    plsc.subcore_barrier()
    def body(j, carry):
        pltpu.sync_copy(ones_v, acc_s.at[dst_v.at[j]], add=True)
        return carry
    lax.fori_loop(0, _NCHUNK, body, 0)
    plsc.subcore_barrier()
    @pl.when(sid < 10)
    def _dump():
        pltpu.sync_copy(acc_s.at[pl.ds(sid * 1000, 1000)],
                        out_hbm.at[pl.ds(cid * _N + sid * 1000, 1000)])


@functools.partial(
    pl.kernel,
    out_type=jax.ShapeDtypeStruct((_NC * _N, _D), jnp.float32),
    mesh=_mesh,
    scratch_types=[
        pltpu.VMEM((_EPWP,), jnp.int32),
        pltpu.VMEM((_NCHUNK, _CH), jnp.int32),
        pltpu.VMEM((_CH, _D), jnp.float32),
        pltpu.VMEM((_CH, _D), jnp.float32),
        pltpu.VMEM_SHARED((_NR, _D), jnp.float32),
        pltpu.SemaphoreType.DMA,
        pltpu.SemaphoreType.DMA,
    ],
)
def _agg_kernel(xs_hbm, src_hbm, dst_hbm, out_hbm,
                src_v, dst_v, buf0, buf1, acc_s, sem0, sem1):
    cid = lax.axis_index("c")
    sid = lax.axis_index("s")
    wid = sid * _NC + cid
    def zrow(i, carry):
        for k in range(_D // 16):
            buf0[i, pl.ds(k * 16, 16)] = jnp.zeros((16,), jnp.float32)
        return carry
    lax.fori_loop(0, _CH, zrow, 0)
    for t in range(pl.cdiv(_NZFULL, _NS)):
        c = sid + t * _NS
        @pl.when(c < _NZFULL)
        def _zero():
            pltpu.sync_copy(buf0, acc_s.at[pl.ds(c * _CH, _CH)])
    pltpu.sync_copy(src_hbm.at[wid], src_v)
    pltpu.sync_copy(dst_hbm.at[wid], dst_v)
    plsc.subcore_barrier()
    def src_at(j):
        return src_v.at[pl.ds(j * _CH, _CH)]

    pltpu.async_copy(xs_hbm.at[src_at(0)], buf0, sem0)

    def body(jj, carry):
        j0 = 2 * jj
        pltpu.make_async_copy(xs_hbm.at[src_at(j0)], buf0, sem0).wait()
        @pl.when(j0 + 1 < _NCHUNK)
        def _g1():
            pltpu.async_copy(xs_hbm.at[src_at(j0 + 1)], buf1, sem1)
        pltpu.sync_copy(buf0, acc_s.at[dst_v.at[j0]], add=True)
        @pl.when(j0 + 1 < _NCHUNK)
        def _s1():
            pltpu.make_async_copy(xs_hbm.at[src_at(j0 + 1)], buf1, sem1).wait()
            @pl.when(j0 + 2 < _NCHUNK)
            def _g0():
                pltpu.async_copy(xs_hbm.at[src_at(j0 + 2)], buf0, sem0)
            pltpu.sync_copy(buf1, acc_s.at[dst_v.at[j0 + 1]], add=True)
        return carry
    lax.fori_loop(0, pl.cdiv(_NCHUNK, 2), body, 0)
    plsc.subcore_barrier()
    @pl.when(sid < 10)
    def _dump():
        pltpu.sync_copy(acc_s.at[pl.ds(sid * 1000, 1000)],
                        out_hbm.at[pl.ds(cid * _N + sid * 1000, 1000)])


def _tc_first(x_ref, w_ref, deg_ref, xs_ref, dis_ref):
    deg = deg_ref[0, :, 0:1] + deg_ref[1, :, 0:1]
    dis = jnp.where(deg > 0, lax.rsqrt(deg), 0.0)
    dis_ref[...] = dis
    xw = jnp.dot(x_ref[...], w_ref[...], preferred_element_type=jnp.float32)
    xs_ref[...] = dis * xw


def _post_layer(agg_ref, dis, b, gamma, beta, h_prev):
    s = agg_ref[0] + agg_ref[1]
    pre = dis * s + b[None, :]
    mu = jnp.mean(pre, axis=0, keepdims=True)
    var = jnp.mean((pre - mu) ** 2, axis=0, keepdims=True)
    hb = gamma[None, :] * (pre - mu) / jnp.sqrt(var + _EPS) + beta[None, :]
    return jnp.maximum(hb, 0.0) + h_prev


def _tc_mid(agg_ref, dis_ref, b_ref, g_ref, bt_ref, h_ref, w_ref,
            h_out_ref, xs_ref):
    dis = dis_ref[...]
    h_new = _post_layer(agg_ref, dis, b_ref[...], g_ref[...], bt_ref[...],
                        h_ref[...])
    h_out_ref[...] = h_new
    xw = jnp.dot(h_new, w_ref[...], preferred_element_type=jnp.float32)
    xs_ref[...] = dis * xw


def _tc_last(agg_ref, dis_ref, b_ref, g_ref, bt_ref, h_ref,
             rw0_ref, rb0_ref, rw1_ref, rb1_ref, rw2_ref, rb2_ref, y_ref):
    h_new = _post_layer(agg_ref, dis_ref[...], b_ref[...], g_ref[...],
                        bt_ref[...], h_ref[...])
    g = jnp.mean(h_new, axis=0, keepdims=True)
    y = jnp.dot(g, rw0_ref[...], preferred_element_type=jnp.float32)
    y = jnp.maximum(y + rb0_ref[...][None, :], 0.0)
    y = jnp.dot(y, rw1_ref[...], preferred_element_type=jnp.float32)
    y = jnp.maximum(y + rb1_ref[...][None, :], 0.0)
    y = jnp.dot(y, rw2_ref[...], preferred_element_type=jnp.float32)
    y_ref[...] = y + rb2_ref[...][None, :]


_f32 = jnp.float32


def kernel(x, edge_index, W, b, gamma, beta, rW0, rb0, rW1, rb1, rW2, rb2):
    src2 = edge_index[0].reshape(_NW, _EPW)
    dst3 = edge_index[1].reshape(_NW, _NCHUNK, _CH)

    deg2 = _deg_kernel(dst3).reshape(_NC, _N, _D)

    xs, dis = pl.pallas_call(
        _tc_first,
        out_shape=(jax.ShapeDtypeStruct((_N, _D), _f32),
                   jax.ShapeDtypeStruct((_N, 1), _f32)),
    )(x, W[0], deg2)

    h = x
    for i in range(_DEPTH - 1):
        agg2 = _agg_kernel(xs, src2, dst3).reshape(_NC, _N, _D)
        h, xs = pl.pallas_call(
            _tc_mid,
            out_shape=(jax.ShapeDtypeStruct((_N, _D), _f32),
                       jax.ShapeDtypeStruct((_N, _D), _f32)),
        )(agg2, dis, b[i], gamma[i], beta[i], h, W[i + 1])

    agg2 = _agg_kernel(xs, src2, dst3).reshape(_NC, _N, _D)
    y = pl.pallas_call(
        _tc_last,
        out_shape=jax.ShapeDtypeStruct((1, len(rb2)), _f32),
    )(agg2, dis, b[_DEPTH - 1], gamma[_DEPTH - 1], beta[_DEPTH - 1], h,
      rW0, rb0, rW1, rb1, rW2, rb2)
    return y.reshape(-1)


# speedup vs baseline: 1.4365x; 1.0051x over previous
import functools

import jax
import jax.numpy as jnp
from jax import lax
from jax.experimental import pallas as pl
from jax.experimental.pallas import tpu as pltpu
from jax.experimental.pallas import tpu_sc as plsc

_N = 10000
_E = 320000
_D = 128
_DEPTH = 4
_EPS = 1e-5

_NC = 2
_NS = 16
_NW = _NC * _NS
_EPW = _E // _NW
_CH = 80
_NCHUNK = _EPW // _CH
_EPWP = _NCHUNK * _CH
_NR = _N
_NZFULL = _N // _CH

_mesh = plsc.VectorSubcoreMesh(core_axis_name="c", subcore_axis_name="s")

@functools.partial(
    pl.kernel,
    out_type=jax.ShapeDtypeStruct((_NC * _N, _D), jnp.float32),
    mesh=_mesh,
    scratch_types=[
        pltpu.VMEM((_NCHUNK, _CH), jnp.int32),
        pltpu.VMEM((_CH, _D), jnp.float32),
        pltpu.VMEM((_CH, _D), jnp.float32),
        pltpu.VMEM_SHARED((_NR, _D), jnp.float32),
    ],
)
def _deg_kernel(dst_hbm, out_hbm, dst_v, ones_v, zeros_v, acc_s):
    cid = lax.axis_index("c")
    sid = lax.axis_index("s")
    wid = sid * _NC + cid

    def fill(i, carry):
        for k in range(_D // 16):
            ones_v[i, pl.ds(k * 16, 16)] = jnp.ones((16,), jnp.float32)
            zeros_v[i, pl.ds(k * 16, 16)] = jnp.zeros((16,), jnp.float32)
        return carry
    lax.fori_loop(0, _CH, fill, 0)
    for t in range(pl.cdiv(_NZFULL, _NS)):
        c = sid + t * _NS
        @pl.when(c < _NZFULL)
        def _zero():
            pltpu.sync_copy(zeros_v, acc_s.at[pl.ds(c * _CH, _CH)])
    pltpu.sync_copy(dst_hbm.at[wid], dst_v)
    plsc.subcore_barrier()
    def body(j, carry):
        pltpu.sync_copy(ones_v, acc_s.at[dst_v.at[j]], add=True)
        return carry
    lax.fori_loop(0, _NCHUNK, body, 0)
    plsc.subcore_barrier()
    @pl.when(sid < 10)
    def _dump():
        pltpu.sync_copy(acc_s.at[pl.ds(sid * 1000, 1000)],
                        out_hbm.at[pl.ds(cid * _N + sid * 1000, 1000)])


@functools.partial(
    pl.kernel,
    out_type=jax.ShapeDtypeStruct((_NC * _N, _D), jnp.float32),
    mesh=_mesh,
    scratch_types=[
        pltpu.VMEM((_EPWP,), jnp.int32),
        pltpu.VMEM((_NCHUNK, _CH), jnp.int32),
        pltpu.VMEM((_CH, _D), jnp.float32),
        pltpu.VMEM((_CH, _D), jnp.float32),
        pltpu.VMEM_SHARED((_NR, _D), jnp.float32),
        pltpu.SemaphoreType.DMA,
        pltpu.SemaphoreType.DMA,
        pltpu.SemaphoreType.DMA,
        pltpu.SemaphoreType.DMA,
    ],
)
def _agg_kernel(xs_hbm, src_hbm, dst_hbm, out_hbm,
                src_v, dst_v, buf0, buf1, acc_s, semg0, semg1, sems0, sems1):
    cid = lax.axis_index("c")
    sid = lax.axis_index("s")
    wid = sid * _NC + cid
    def zrow(i, carry):
        for k in range(_D // 16):
            buf0[i, pl.ds(k * 16, 16)] = jnp.zeros((16,), jnp.float32)
        return carry
    lax.fori_loop(0, _CH, zrow, 0)
    for t in range(pl.cdiv(_NZFULL, _NS)):
        c = sid + t * _NS
        @pl.when(c < _NZFULL)
        def _zero():
            pltpu.sync_copy(buf0, acc_s.at[pl.ds(c * _CH, _CH)])
    pltpu.sync_copy(src_hbm.at[wid], src_v)
    pltpu.sync_copy(dst_hbm.at[wid], dst_v)
    plsc.subcore_barrier()
    def src_at(j):
        return src_v.at[pl.ds(j * _CH, _CH)]

    pltpu.async_copy(xs_hbm.at[src_at(0)], buf0, semg0)
    pltpu.async_copy(xs_hbm.at[src_at(1)], buf1, semg1)

    def body(jj, carry):
        j0 = 2 * jj
        pltpu.make_async_copy(xs_hbm.at[src_at(j0)], buf0, semg0).wait()
        pltpu.async_copy(buf0, acc_s.at[dst_v.at[j0]], sems0, add=True)
        @pl.when(j0 + 1 < _NCHUNK)
        def _b1():
            pltpu.make_async_copy(xs_hbm.at[src_at(j0 + 1)], buf1,
                                  semg1).wait()
            pltpu.async_copy(buf1, acc_s.at[dst_v.at[j0 + 1]], sems1,
                             add=True)
        @pl.when(j0 + 2 < _NCHUNK)
        def _g0():
            pltpu.make_async_copy(buf0, acc_s.at[dst_v.at[j0]],
                                  sems0).wait()
            pltpu.async_copy(xs_hbm.at[src_at(j0 + 2)], buf0, semg0)
        @pl.when(j0 + 3 < _NCHUNK)
        def _g1():
            pltpu.make_async_copy(buf1, acc_s.at[dst_v.at[j0 + 1]],
                                  sems1).wait()
            pltpu.async_copy(xs_hbm.at[src_at(j0 + 3)], buf1, semg1)
        return carry
    lax.fori_loop(0, pl.cdiv(_NCHUNK, 2), body, 0)
    pltpu.make_async_copy(buf0, acc_s.at[dst_v.at[_NCHUNK - 1]],
                          sems0).wait()
    pltpu.make_async_copy(buf1, acc_s.at[dst_v.at[_NCHUNK - 2]],
                          sems1).wait()
    plsc.subcore_barrier()
    @pl.when(sid < 10)
    def _dump():
        pltpu.sync_copy(acc_s.at[pl.ds(sid * 1000, 1000)],
                        out_hbm.at[pl.ds(cid * _N + sid * 1000, 1000)])


def _tc_first(x_ref, w_ref, deg_ref, xs_ref, dis_ref):
    deg = deg_ref[0, :, 0:1] + deg_ref[1, :, 0:1]
    dis = jnp.where(deg > 0, lax.rsqrt(deg), 0.0)
    dis_ref[...] = dis
    xw = jnp.dot(x_ref[...], w_ref[...], preferred_element_type=jnp.float32)
    xs_ref[...] = dis * xw


def _post_layer(agg_ref, dis, b, gamma, beta, h_prev):
    s = agg_ref[0] + agg_ref[1]
    pre = dis * s + b[None, :]
    mu = jnp.mean(pre, axis=0, keepdims=True)
    var = jnp.mean((pre - mu) ** 2, axis=0, keepdims=True)
    hb = gamma[None, :] * (pre - mu) / jnp.sqrt(var + _EPS) + beta[None, :]
    return jnp.maximum(hb, 0.0) + h_prev


def _tc_mid(agg_ref, dis_ref, b_ref, g_ref, bt_ref, h_ref, w_ref,
            h_out_ref, xs_ref):
    dis = dis_ref[...]
    h_new = _post_layer(agg_ref, dis, b_ref[...], g_ref[...], bt_ref[...],
                        h_ref[...])
    h_out_ref[...] = h_new
    xw = jnp.dot(h_new, w_ref[...], preferred_element_type=jnp.float32)
    xs_ref[...] = dis * xw


def _tc_last(agg_ref, dis_ref, b_ref, g_ref, bt_ref, h_ref,
             rw0_ref, rb0_ref, rw1_ref, rb1_ref, rw2_ref, rb2_ref, y_ref):
    h_new = _post_layer(agg_ref, dis_ref[...], b_ref[...], g_ref[...],
                        bt_ref[...], h_ref[...])
    g = jnp.mean(h_new, axis=0, keepdims=True)
    y = jnp.dot(g, rw0_ref[...], preferred_element_type=jnp.float32)
    y = jnp.maximum(y + rb0_ref[...][None, :], 0.0)
    y = jnp.dot(y, rw1_ref[...], preferred_element_type=jnp.float32)
    y = jnp.maximum(y + rb1_ref[...][None, :], 0.0)
    y = jnp.dot(y, rw2_ref[...], preferred_element_type=jnp.float32)
    y_ref[...] = y + rb2_ref[...][None, :]


_f32 = jnp.float32


def kernel(x, edge_index, W, b, gamma, beta, rW0, rb0, rW1, rb1, rW2, rb2):
    src2 = edge_index[0].reshape(_NW, _EPW)
    dst3 = edge_index[1].reshape(_NW, _NCHUNK, _CH)

    deg2 = _deg_kernel(dst3).reshape(_NC, _N, _D)

    xs, dis = pl.pallas_call(
        _tc_first,
        out_shape=(jax.ShapeDtypeStruct((_N, _D), _f32),
                   jax.ShapeDtypeStruct((_N, 1), _f32)),
    )(x, W[0], deg2)

    h = x
    for i in range(_DEPTH - 1):
        agg2 = _agg_kernel(xs, src2, dst3).reshape(_NC, _N, _D)
        h, xs = pl.pallas_call(
            _tc_mid,
            out_shape=(jax.ShapeDtypeStruct((_N, _D), _f32),
                       jax.ShapeDtypeStruct((_N, _D), _f32)),
        )(agg2, dis, b[i], gamma[i], beta[i], h, W[i + 1])

    agg2 = _agg_kernel(xs, src2, dst3).reshape(_NC, _N, _D)
    y = pl.pallas_call(
        _tc_last,
        out_shape=jax.ShapeDtypeStruct((1, len(rb2)), _f32),
    )(agg2, dis, b[_DEPTH - 1], gamma[_DEPTH - 1], beta[_DEPTH - 1], h,
      rW0, rb0, rW1, rb1, rW2, rb2)
    return y.reshape(-1)
